# Initial kernel scaffold; baseline (speedup 1.0000x reference)
import jax
import jax.numpy as jnp
from jax.experimental import pallas as pl


def kernel(x, params, edge_index):
    raise NotImplementedError("write your pallas kernel here")


# speedup vs baseline: 4.1733x; 4.1733x over previous
import functools

import jax
import jax.numpy as jnp
from jax import lax
from jax.experimental import pallas as pl
from jax.experimental.pallas import tpu as pltpu
from jax.experimental.pallas import tpu_sc as plsc

NN = 10000
EE = 320000
FD = 128
NS = 16
NCORE = 2
KC = 80
NCH = EE // KC
NPT = NCH // (NCORE * NS)
NPAD = 10240
RPT = NPAD // NS

_MESH = plsc.VectorSubcoreMesh(
    core_axis_name="c", subcore_axis_name="s",
    num_cores=NCORE, num_subcores=NS)

_F32 = jnp.float32


def _hi_dot(a, b):
    return jnp.dot(a, b, precision=lax.Precision.HIGHEST,
                   preferred_element_type=_F32)


def _sc_ax2(x, row2, col2, z128):
    @functools.partial(
        pl.kernel,
        out_type=jax.ShapeDtypeStruct((NCORE, NPAD, FD), _F32),
        mesh=_MESH,
        scratch_types=(
            pltpu.VMEM_SHARED((NPAD, FD), _F32),
            pltpu.VMEM((KC,), jnp.int32),
            pltpu.VMEM((KC,), jnp.int32),
            pltpu.VMEM((KC, FD), _F32),
            pltpu.VMEM((KC, FD), _F32),
            pltpu.SemaphoreType.DMA,
        ),
    )
    def k(x_hbm, row2_hbm, col2_hbm, z128_hbm,
          outp_hbm,
          acc, colbuf, rowbuf, xbuf, sqbuf, sem):
        c = lax.axis_index("c")
        s = lax.axis_index("s")
        rbase = s * RPT
        pltpu.sync_copy(z128_hbm.at[pl.ds(0, KC)], xbuf)

        def zbody(i, carry):
            pltpu.sync_copy(xbuf, acc.at[pl.ds(rbase + i * KC, KC)])
            return carry

        lax.fori_loop(0, RPT // KC, zbody, 0)
        plsc.subcore_barrier()

        def chunk(j, carry):
            cj = (c * NS + s) * NPT + j
            pltpu.sync_copy(col2_hbm.at[cj], colbuf)
            pltpu.sync_copy(row2_hbm.at[cj], rowbuf)
            pltpu.async_copy(x_hbm.at[colbuf], xbuf, sem).wait()

            def ebody(e, cc):
                for d in range(FD // 16):
                    v = xbuf[e, pl.ds(d * 16, 16)]
                    sqbuf[e, pl.ds(d * 16, 16)] = v * v
                return cc

            lax.fori_loop(0, KC, ebody, 0)
            pltpu.sync_copy(sqbuf, acc.at[rowbuf], add=True)
            return carry

        lax.fori_loop(0, NPT, chunk, 0)
        plsc.subcore_barrier()

        def wbody(i, carry):
            b = rbase + i * KC
            pltpu.sync_copy(acc.at[pl.ds(b, KC)], xbuf)
            pltpu.sync_copy(xbuf, outp_hbm.at[c, pl.ds(b, KC)])
            return carry

        lax.fori_loop(0, RPT // KC, wbody, 0)

    return k(x, row2, col2, z128)


def _sc_segsum(h, row2, col2, z128):
    @functools.partial(
        pl.kernel,
        out_type=jax.ShapeDtypeStruct((NCORE, NPAD, FD), _F32),
        mesh=_MESH,
        scratch_types=(
            pltpu.VMEM_SHARED((NPAD, FD), _F32),
            pltpu.VMEM((KC,), jnp.int32),
            pltpu.VMEM((KC,), jnp.int32),
            pltpu.VMEM((KC, FD), _F32),
            pltpu.SemaphoreType.DMA,
        ),
    )
    def k(h_hbm, row2_hbm, col2_hbm, z128_hbm,
          outp_hbm,
          acc, colbuf, rowbuf, hbuf, sem):
        c = lax.axis_index("c")
        s = lax.axis_index("s")
        rbase = s * RPT
        pltpu.sync_copy(z128_hbm.at[pl.ds(0, KC)], hbuf)

        def zbody(i, carry):
            pltpu.sync_copy(hbuf, acc.at[pl.ds(rbase + i * KC, KC)])
            return carry

        lax.fori_loop(0, RPT // KC, zbody, 0)
        plsc.subcore_barrier()

        def chunk(j, carry):
            cj = (c * NS + s) * NPT + j
            pltpu.sync_copy(col2_hbm.at[cj], colbuf)
            pltpu.sync_copy(row2_hbm.at[cj], rowbuf)
            pltpu.async_copy(h_hbm.at[colbuf], hbuf, sem).wait()
            pltpu.sync_copy(hbuf, acc.at[rowbuf], add=True)
            return carry

        lax.fori_loop(0, NPT, chunk, 0)
        plsc.subcore_barrier()

        def wbody(i, carry):
            b = rbase + i * KC
            pltpu.sync_copy(acc.at[pl.ds(b, KC)], hbuf)
            pltpu.sync_copy(hbuf, outp_hbm.at[c, pl.ds(b, KC)])
            return carry

        lax.fori_loop(0, RPT // KC, wbody, 0)

    return k(h, row2, col2, z128)


def _sc_es_segsum(es128, row2, z128):
    @functools.partial(
        pl.kernel,
        out_type=jax.ShapeDtypeStruct((NCORE, NPAD, FD), _F32),
        mesh=_MESH,
        scratch_types=(
            pltpu.VMEM_SHARED((NPAD, FD), _F32),
            pltpu.VMEM((KC,), jnp.int32),
            pltpu.VMEM((KC, FD), _F32),
        ),
    )
    def k(es_hbm, row2_hbm, z128_hbm,
          outp_hbm,
          acc, rowbuf, ebuf):
        c = lax.axis_index("c")
        s = lax.axis_index("s")
        rbase = s * RPT
        pltpu.sync_copy(z128_hbm.at[pl.ds(0, KC)], ebuf)

        def zbody(i, carry):
            pltpu.sync_copy(ebuf, acc.at[pl.ds(rbase + i * KC, KC)])
            return carry

        lax.fori_loop(0, RPT // KC, zbody, 0)
        plsc.subcore_barrier()

        def chunk(j, carry):
            cj = (c * NS + s) * NPT + j
            pltpu.sync_copy(row2_hbm.at[cj], rowbuf)
            pltpu.sync_copy(es_hbm.at[pl.ds(cj * KC, KC)], ebuf)
            pltpu.sync_copy(ebuf, acc.at[rowbuf], add=True)
            return carry

        lax.fori_loop(0, NPT, chunk, 0)
        plsc.subcore_barrier()

        def wbody(i, carry):
            b = rbase + i * KC
            pltpu.sync_copy(acc.at[pl.ds(b, KC)], ebuf)
            pltpu.sync_copy(ebuf, outp_hbm.at[c, pl.ds(b, KC)])
            return carry

        lax.fori_loop(0, RPT // KC, wbody, 0)

    return k(es128, row2, z128)


def _sc_deg(row2, z128, e1):
    @functools.partial(
        pl.kernel,
        out_type=jax.ShapeDtypeStruct((NCORE, NPAD, FD), _F32),
        mesh=_MESH,
        scratch_types=(
            pltpu.VMEM_SHARED((NPAD, FD), _F32),
            pltpu.VMEM((KC,), jnp.int32),
            pltpu.VMEM((KC, FD), _F32),
            pltpu.VMEM((KC, FD), _F32),
        ),
    )
    def k(row2_hbm, z128_hbm, e1_hbm,
          outp_hbm,
          acc, rowbuf, ebuf, onesbuf):
        c = lax.axis_index("c")
        s = lax.axis_index("s")
        rbase = s * RPT
        pltpu.sync_copy(z128_hbm.at[pl.ds(0, KC)], ebuf)
        pltpu.sync_copy(e1_hbm, onesbuf)

        def zbody(i, carry):
            pltpu.sync_copy(ebuf, acc.at[pl.ds(rbase + i * KC, KC)])
            return carry

        lax.fori_loop(0, RPT // KC, zbody, 0)
        plsc.subcore_barrier()

        def chunk(j, carry):
            cj = (c * NS + s) * NPT + j
            pltpu.sync_copy(row2_hbm.at[cj], rowbuf)
            pltpu.sync_copy(onesbuf, acc.at[rowbuf], add=True)
            return carry

        lax.fori_loop(0, NPT, chunk, 0)
        plsc.subcore_barrier()

        def wbody(i, carry):
            b = rbase + i * KC
            pltpu.sync_copy(acc.at[pl.ds(b, KC)], ebuf)
            pltpu.sync_copy(ebuf, outp_hbm.at[c, pl.ds(b, KC)])
            return carry

        lax.fori_loop(0, RPT // KC, wbody, 0)

    return k(row2, z128, e1)


def _sc_attn(q, kk, v, row2, col2, z128):
    @functools.partial(
        pl.kernel,
        out_type=(
            jax.ShapeDtypeStruct((NCORE, NPAD, FD), _F32),
            jax.ShapeDtypeStruct((EE, FD), _F32),
        ),
        mesh=_MESH,
        scratch_types=(
            pltpu.VMEM_SHARED((NPAD, FD), _F32),
            pltpu.VMEM((KC,), jnp.int32),
            pltpu.VMEM((KC,), jnp.int32),
            pltpu.VMEM((KC, FD), _F32),
            pltpu.VMEM((KC, FD), _F32),
            pltpu.VMEM((KC, FD), _F32),
            pltpu.VMEM((KC, FD), _F32),
            pltpu.SemaphoreType.DMA,
        ),
    )
    def k(q_hbm, k_hbm, v_hbm, row2_hbm, col2_hbm, z128_hbm,
          nump_hbm, es_hbm,
          acc, colbuf, rowbuf, qbuf, kbuf, vbuf, dbuf, sem):
        c = lax.axis_index("c")
        s = lax.axis_index("s")
        rbase = s * RPT
        pltpu.sync_copy(z128_hbm.at[pl.ds(0, KC)], qbuf)
        pltpu.sync_copy(z128_hbm.at[pl.ds(0, KC)], dbuf)

        def zbody(i, carry):
            pltpu.sync_copy(qbuf, acc.at[pl.ds(rbase + i * KC, KC)])
            return carry

        lax.fori_loop(0, RPT // KC, zbody, 0)
        plsc.subcore_barrier()

        inv_sqrt = 1.0 / (FD ** 0.5)
        lane = lax.iota(jnp.int32, 16)
        dnums = lax.GatherDimensionNumbers(
            offset_dims=(), collapsed_slice_dims=(0,), start_index_map=(0,))

        def chunk(j, carry):
            cj = (c * NS + s) * NPT + j
            pltpu.sync_copy(col2_hbm.at[cj], colbuf)
            pltpu.sync_copy(row2_hbm.at[cj], rowbuf)
            d1 = pltpu.async_copy(q_hbm.at[rowbuf], qbuf, sem)
            d2 = pltpu.async_copy(k_hbm.at[colbuf], kbuf, sem)
            d3 = pltpu.async_copy(v_hbm.at[colbuf], vbuf, sem)
            d1.wait()
            d2.wait()
            d3.wait()

            def ebody(e, cc):
                dot = qbuf[e, pl.ds(0, 16)] * kbuf[e, pl.ds(0, 16)]
                for d in range(1, FD // 16):
                    dot = dot + (qbuf[e, pl.ds(d * 16, 16)]
                                 * kbuf[e, pl.ds(d * 16, 16)])
                for sh in (8, 4, 2, 1):
                    perm = lax.gather(dot, (lane ^ sh)[:, None], dnums, (1,),
                                      mode=lax.GatherScatterMode.PROMISE_IN_BOUNDS)
                    dot = dot + perm
                sv = dot * inv_sqrt
                sv = jnp.where(sv > 0, sv, 0.2 * sv)
                esv = jnp.exp(sv)
                for d in range(FD // 16):
                    vbuf[e, pl.ds(d * 16, 16)] = vbuf[e, pl.ds(d * 16, 16)] * esv
                dbuf[e, pl.ds(0, 16)] = esv
                return cc

            lax.fori_loop(0, KC, ebody, 0)
            pltpu.sync_copy(vbuf, acc.at[rowbuf], add=True)
            pltpu.sync_copy(dbuf, es_hbm.at[pl.ds(cj * KC, KC)])
            return carry

        lax.fori_loop(0, NPT, chunk, 0)
        plsc.subcore_barrier()

        def wbody(i, carry):
            b = rbase + i * KC
            pltpu.sync_copy(acc.at[pl.ds(b, KC)], qbuf)
            pltpu.sync_copy(qbuf, nump_hbm.at[c, pl.ds(b, KC)])
            return carry

        lax.fori_loop(0, RPT // KC, wbody, 0)

    return k(q, kk, v, row2, col2, z128)


GB = 10
BR = NN // GB


def _bs(shape, imap):
    return pl.BlockSpec(shape, imap)


def _full(shape):
    return pl.BlockSpec(shape, lambda g, _s=len(shape): (0,) * _s)


def _tc_a(axp, ax2p, degp, p):
    def body(axp_r, ax2p_r, degp_r, w_r, b_r, ax_o, ax2_o, deg_o, u_o, ps_o):
        AX = axp_r[0] + axp_r[1]
        AX2 = ax2p_r[0] + ax2p_r[1]
        ax_o[...] = AX
        ax2_o[...] = AX2
        deg = degp_r[0, :, 0:1] + degp_r[1, :, 0:1]
        deg_o[...] = jnp.broadcast_to(deg, (BR, 16))
        U = _hi_dot(AX, w_r[...]) + deg * b_r[...]
        u_o[...] = U
        s1 = jnp.sum(U, axis=0)
        s2 = jnp.sum(U * U, axis=0)
        z = jnp.zeros((14, 2 * FD), _F32)
        ps_o[...] = jnp.concatenate([s1[None], s2[None], z], axis=0)[None]

    return pl.pallas_call(
        body,
        grid=(GB,),
        in_specs=[
            _bs((NCORE, BR, FD), lambda g: (0, g, 0)),
            _bs((NCORE, BR, FD), lambda g: (0, g, 0)),
            _bs((NCORE, BR, FD), lambda g: (0, g, 0)),
            _full((FD, 2 * FD)),
            _full((2 * FD,)),
        ],
        out_specs=[
            _bs((BR, FD), lambda g: (g, 0)),
            _bs((BR, FD), lambda g: (g, 0)),
            _bs((BR, 16), lambda g: (g, 0)),
            _bs((BR, 2 * FD), lambda g: (g, 0)),
            _bs((1, 16, 2 * FD), lambda g: (g, 0, 0)),
        ],
        out_shape=(
            jax.ShapeDtypeStruct((NN, FD), _F32),
            jax.ShapeDtypeStruct((NN, FD), _F32),
            jax.ShapeDtypeStruct((NN, 16), _F32),
            jax.ShapeDtypeStruct((NN, 2 * FD), _F32),
            jax.ShapeDtypeStruct((GB, 16, 2 * FD), _F32),
        ),
    )(axp[:, :NN], ax2p[:, :NN], degp[:, :NN], p['gc1_W'], p['gc1_b'])


def _tc_stats(ps, width):
    def body(ps_r, mv_o):
        s1 = jnp.sum(ps_r[:, 0, :], axis=0)
        s2 = jnp.sum(ps_r[:, 1, :], axis=0)
        m = s1 / NN
        v = s2 / NN - m * m
        z = jnp.zeros((14, width), _F32)
        mv_o[...] = jnp.concatenate([m[None], v[None], z], axis=0)

    return pl.pallas_call(
        body,
        out_shape=jax.ShapeDtypeStruct((16, width), _F32),
    )(ps)


def _tc_b(u, mv, x, ax, ax2, deg16, p):
    def body(u_r, mv_r, x_r, ax_r, ax2_r, deg_r, bn1g_r, bn1b_r,
             fc1w_r, fc1b_r, ln1g_r, ln1b_r, fc2w_r, fc2b_r, xpw_r, xpb_r,
             h1a_o, h1b_o, gate_o, xproj_o):
        U = u_r[...]
        m = mv_r[0]
        v = mv_r[1]
        h1 = jax.nn.relu((U - m) / jnp.sqrt(v + 1e-5) * bn1g_r[...] + bn1b_r[...])
        h1a_o[...] = h1[:, :FD]
        h1b_o[...] = h1[:, FD:]
        xv = x_r[...]
        AX = ax_r[...]
        AX2 = ax2_r[...]
        deg = deg_r[:, 0:1]
        delta1 = AX - xv
        x_mean = AX / (deg + 1e-8)
        varg = AX2 / (deg + 1e-8) - x_mean * x_mean
        x_std = jnp.sqrt(jnp.clip(varg, 0.0, None))
        fw = fc1w_r[...]
        t = (_hi_dot(xv, fw[0:FD, :]) + _hi_dot(delta1, fw[FD:2 * FD, :])
             + _hi_dot(x_std, fw[2 * FD:3 * FD, :]) + deg * fw[3 * FD, :]
             + fc1b_r[...])
        mm = jnp.mean(t, axis=-1, keepdims=True)
        vv = jnp.var(t, axis=-1, keepdims=True)
        g = jax.nn.relu((t - mm) / jnp.sqrt(vv + 1e-5) * ln1g_r[...] + ln1b_r[...])
        logits = _hi_dot(g, fc2w_r[...]) + fc2b_r[...]
        gate_o[...] = jax.nn.softmax(logits * 0.5, axis=1)
        xproj_o[...] = _hi_dot(xv, xpw_r[...]) + xpb_r[...]

    return pl.pallas_call(
        body,
        grid=(GB,),
        in_specs=[
            _bs((BR, 2 * FD), lambda g: (g, 0)),
            _full((16, 2 * FD)),
            _bs((BR, FD), lambda g: (g, 0)),
            _bs((BR, FD), lambda g: (g, 0)),
            _bs((BR, FD), lambda g: (g, 0)),
            _bs((BR, 16), lambda g: (g, 0)),
            _full((2 * FD,)),
            _full((2 * FD,)),
            _full((3 * FD + 1, FD)),
            _full((FD,)),
            _full((FD,)),
            _full((FD,)),
            _full((FD, 2)),
            _full((2,)),
            _full((FD, FD)),
            _full((FD,)),
        ],
        out_specs=[
            _bs((BR, FD), lambda g: (g, 0)),
            _bs((BR, FD), lambda g: (g, 0)),
            _bs((BR, 2), lambda g: (g, 0)),
            _bs((BR, FD), lambda g: (g, 0)),
        ],
        out_shape=(
            jax.ShapeDtypeStruct((NN, FD), _F32),
            jax.ShapeDtypeStruct((NN, FD), _F32),
            jax.ShapeDtypeStruct((NN, 2), _F32),
            jax.ShapeDtypeStruct((NN, FD), _F32),
        ),
    )(u, mv, x, ax, ax2, deg16, p['bn1_g'], p['bn1_b'], p['fc1_W'],
      p['fc1_b'], p['ln1_g'], p['ln1_b'], p['fc2_W'], p['fc2_b'],
      p['xproj_W'], p['xproj_b'])


def _tc_c(ahap, ahbp, deg16, p):
    def body(ahap_r, ahbp_r, deg_r, w_r, b_r, u_o, ps_o):
        aha = ahap_r[0] + ahap_r[1]
        ahb = ahbp_r[0] + ahbp_r[1]
        deg = deg_r[:, 0:1]
        w2 = w_r[...]
        U = _hi_dot(aha, w2[:FD, :]) + _hi_dot(ahb, w2[FD:, :]) + deg * b_r[...]
        u_o[...] = U
        s1 = jnp.sum(U, axis=0)
        s2 = jnp.sum(U * U, axis=0)
        z = jnp.zeros((14, FD), _F32)
        ps_o[...] = jnp.concatenate([s1[None], s2[None], z], axis=0)[None]

    return pl.pallas_call(
        body,
        grid=(GB,),
        in_specs=[
            _bs((NCORE, BR, FD), lambda g: (0, g, 0)),
            _bs((NCORE, BR, FD), lambda g: (0, g, 0)),
            _bs((BR, 16), lambda g: (g, 0)),
            _full((2 * FD, FD)),
            _full((FD,)),
        ],
        out_specs=[
            _bs((BR, FD), lambda g: (g, 0)),
            _bs((1, 16, FD), lambda g: (g, 0, 0)),
        ],
        out_shape=(
            jax.ShapeDtypeStruct((NN, FD), _F32),
            jax.ShapeDtypeStruct((GB, 16, FD), _F32),
        ),
    )(ahap[:, :NN], ahbp[:, :NN], deg16, p['gc2_W'], p['gc2_b'])


def _tc_d(u2, mv2, xproj, rs11, p):
    def body(u_r, mv_r, xp_r, rs_r, bn2g_r, bn2b_r, rng_r, rnb_r,
             wq_r, wk_r, wv_r, low_o, q_o, k_o, v_o):
        U = u_r[...]
        m = mv_r[0]
        v = mv_r[1]
        low = jax.nn.relu((U - m) / jnp.sqrt(v + 1e-5) * bn2g_r[...] + bn2b_r[...])
        low_o[...] = low
        r = rs_r[0, 0] * (xp_r[...] - low)
        mm = jnp.mean(r, axis=-1, keepdims=True)
        vv = jnp.var(r, axis=-1, keepdims=True)
        resid = (r - mm) / jnp.sqrt(vv + 1e-5) * rng_r[...] + rnb_r[...]
        q_o[...] = _hi_dot(resid, wq_r[...])
        k_o[...] = _hi_dot(resid, wk_r[...])
        v_o[...] = _hi_dot(resid, wv_r[...])

    return pl.pallas_call(
        body,
        grid=(GB,),
        in_specs=[
            _bs((BR, FD), lambda g: (g, 0)),
            _full((16, FD)),
            _bs((BR, FD), lambda g: (g, 0)),
            _full((1, 1)),
            _full((FD,)),
            _full((FD,)),
            _full((FD,)),
            _full((FD,)),
            _full((FD, FD)),
            _full((FD, FD)),
            _full((FD, FD)),
        ],
        out_specs=[
            _bs((BR, FD), lambda g: (g, 0)),
            _bs((BR, FD), lambda g: (g, 0)),
            _bs((BR, FD), lambda g: (g, 0)),
            _bs((BR, FD), lambda g: (g, 0)),
        ],
        out_shape=(
            jax.ShapeDtypeStruct((NN, FD), _F32),
            jax.ShapeDtypeStruct((NN, FD), _F32),
            jax.ShapeDtypeStruct((NN, FD), _F32),
            jax.ShapeDtypeStruct((NN, FD), _F32),
        ),
    )(u2, mv2, xproj, rs11, p['bn2_g'], p['bn2_b'], p['rn_g'], p['rn_b'],
      p['Wq'], p['Wk'], p['Wv'])


def _tc_e(nump, denp, low, gate):
    def body(nump_r, denp_r, low_r, gate_r, out_o, high_o):
        numer = nump_r[0] + nump_r[1]
        den = denp_r[0, :, 0:1] + denp_r[1, :, 0:1]
        high = numer / (den + 1e-16)
        high_o[...] = high
        out_o[...] = gate_r[:, 0:1] * low_r[...] + gate_r[:, 1:2] * high

    return pl.pallas_call(
        body,
        grid=(GB,),
        in_specs=[
            _bs((NCORE, BR, FD), lambda g: (0, g, 0)),
            _bs((NCORE, BR, FD), lambda g: (0, g, 0)),
            _bs((BR, FD), lambda g: (g, 0)),
            _bs((BR, 2), lambda g: (g, 0)),
        ],
        out_specs=[
            _bs((BR, FD), lambda g: (g, 0)),
            _bs((BR, FD), lambda g: (g, 0)),
        ],
        out_shape=(
            jax.ShapeDtypeStruct((NN, FD), _F32),
            jax.ShapeDtypeStruct((NN, FD), _F32),
        ),
    )(nump[:, :NN], denp[:, :NN], low, gate)


def kernel(x, params, edge_index):
    row2 = edge_index[0].reshape(NCH, KC)
    col2 = edge_index[1].reshape(NCH, KC)
    z128 = jnp.zeros((RPT, FD), _F32)
    rs11 = params['res_scale'].reshape(1, 1)

    axp = _sc_segsum(x, row2, col2, z128)
    ax2p = _sc_ax2(x, row2, col2, z128)
    e1 = jnp.zeros((KC, FD), _F32).at[:, 0].set(1.0)
    degp = _sc_deg(row2, z128, e1)
    ax, ax2, deg16, u1, ps1 = _tc_a(axp, ax2p, degp, params)
    mv1 = _tc_stats(ps1, 2 * FD)
    h1a, h1b, gate, xproj = _tc_b(u1, mv1, x, ax, ax2, deg16, params)
    ahap = _sc_segsum(h1a, row2, col2, z128)
    ahbp = _sc_segsum(h1b, row2, col2, z128)
    u2, ps2 = _tc_c(ahap, ahbp, deg16, params)
    mv2 = _tc_stats(ps2, FD)
    low, q, kk, v = _tc_d(u2, mv2, xproj, rs11, params)
    nump, es16 = _sc_attn(q, kk, v, row2, col2, z128)
    denp = _sc_es_segsum(es16, row2, z128)
    out, high = _tc_e(nump, denp, low, gate)
    return out, gate, low, high
